# baseline (speedup 1.0000x reference)
import jax
import jax.numpy as jnp
from jax.experimental import pallas as pl
from jax.experimental.pallas import tpu as pltpu

B = 128
IN_FEAT = 256
OUT_FEAT = 128
N = 38
EMBED_DIM = 128
ALPHA = 0.2
TOP_K = 10

BB = 32


def _gat_kernel(xt_ref, w_ref, a_ref, emb_ref, out_ref, maskt_ref):
    @pl.when(pl.program_id(0) == 0)
    def _():
        emb = emb_ref[...]
        gram = jax.lax.dot_general(
            emb, emb, (((1,), (1,)), ((), ())),
            preferred_element_type=jnp.float32)
        nrm = jnp.sqrt(jnp.sum(emb * emb, axis=1, keepdims=True))
        adj = gram / (nrm * nrm.T)
        a1_ = adj[:, None, :]
        a2_ = adj[None, :, :]
        mdx = jax.lax.broadcasted_iota(jnp.int32, (N, N, N), 0)
        kdx = jax.lax.broadcasted_iota(jnp.int32, (N, N, N), 1)
        gt = (a1_ > a2_) | ((a1_ == a2_) & (mdx < kdx))
        rank = jnp.sum(gt.astype(jnp.float32), axis=0)
        sel = (rank == jnp.float32(TOP_K - 2)).astype(jnp.float32)
        thresh_t = jnp.sum(adj * sel, axis=0, keepdims=True)
        maskt = (adj > thresh_t) | (adj == jnp.float32(1.0))
        maskt_ref[...] = maskt.astype(jnp.float32)

    w = w_ref[...]
    a = a_ref[...]
    a1 = a[:OUT_FEAT, :]
    a2 = a[OUT_FEAT:, :]
    maskt = maskt_ref[...] > jnp.float32(0.5)

    hs = [
        jax.lax.dot_general(
            xt_ref[b], w, (((1,), (0,)), ((), ())),
            preferred_element_type=jnp.float32)
        for b in range(BB)
    ]
    f1s = [
        jnp.transpose(jax.lax.dot_general(
            h, a1, (((1,), (0,)), ((), ())),
            preferred_element_type=jnp.float32))
        for h in hs
    ]
    f2s = [
        jax.lax.dot_general(
            h, a2, (((1,), (0,)), ((), ())),
            preferred_element_type=jnp.float32)
        for h in hs
    ]
    atts = []
    for b in range(BB):
        et = f2s[b] + f1s[b]
        et = jnp.maximum(et, jnp.float32(ALPHA) * et)
        att = jnp.where(maskt, et, jnp.float32(-1e12))
        att = att - jnp.max(att, axis=1, keepdims=True)
        att = jnp.exp(att)
        atts.append(att / jnp.sum(att, axis=1, keepdims=True))
    for b in range(BB):
        hp = jax.lax.dot_general(
            hs[b], atts[b], (((0,), (0,)), ((), ())),
            preferred_element_type=jnp.float32)
        out_ref[b] = jnp.where(hp > 0, hp, jnp.exp(hp) - 1.0)


@jax.jit
def kernel(x, W, a, emb):
    xt = jnp.transpose(x, (0, 2, 1))
    grid = (B // BB,)
    return pl.pallas_call(
        _gat_kernel,
        grid=grid,
        in_specs=[
            pl.BlockSpec((BB, N, IN_FEAT), lambda b: (b, 0, 0)),
            pl.BlockSpec((IN_FEAT, OUT_FEAT), lambda b: (0, 0)),
            pl.BlockSpec((2 * OUT_FEAT, 1), lambda b: (0, 0)),
            pl.BlockSpec((N, EMBED_DIM), lambda b: (0, 0)),
        ],
        out_specs=pl.BlockSpec((BB, OUT_FEAT, N), lambda b: (b, 0, 0)),
        out_shape=jax.ShapeDtypeStruct((B, OUT_FEAT, N), jnp.float32),
        scratch_shapes=[pltpu.VMEM((N, N), jnp.float32)],
    )(xt, W, a, emb)


# speedup vs baseline: 1.1777x; 1.1777x over previous
import jax
import jax.numpy as jnp
from jax.experimental import pallas as pl
from jax.experimental.pallas import tpu as pltpu

B = 128
IN_FEAT = 256
OUT_FEAT = 128
N = 38
EMBED_DIM = 128
ALPHA = 0.2
TOP_K = 10

BB = 32


def _gat_kernel(x_ref, w_ref, a_ref, emb_ref, out_ref, maskt_ref):
    @pl.when(pl.program_id(0) == 0)
    def _():
        emb = emb_ref[...]
        gram = jax.lax.dot_general(
            emb, emb, (((1,), (1,)), ((), ())),
            preferred_element_type=jnp.float32)
        nrm = jnp.sqrt(jnp.sum(emb * emb, axis=1, keepdims=True))
        adj = gram / (nrm * nrm.T)
        a1_ = adj[:, None, :]
        a2_ = adj[None, :, :]
        mdx = jax.lax.broadcasted_iota(jnp.int32, (N, N, N), 0)
        kdx = jax.lax.broadcasted_iota(jnp.int32, (N, N, N), 1)
        gt = (a1_ > a2_) | ((a1_ == a2_) & (mdx < kdx))
        rank = jnp.sum(gt.astype(jnp.float32), axis=0)
        sel = (rank == jnp.float32(TOP_K - 2)).astype(jnp.float32)
        thresh_t = jnp.sum(adj * sel, axis=0, keepdims=True)
        maskt = (adj > thresh_t) | (adj == jnp.float32(1.0))
        maskt_ref[...] = maskt.astype(jnp.float32)

    w = w_ref[...]
    a = a_ref[...]
    a1 = a[:OUT_FEAT, :]
    a2 = a[OUT_FEAT:, :]
    maskt = maskt_ref[...] > jnp.float32(0.5)

    xbs = [jnp.transpose(x_ref[b]) for b in range(BB)]
    hts = [
        jax.lax.dot_general(
            w, xb, (((0,), (0,)), ((), ())),
            preferred_element_type=jnp.float32)
        for xb in xbs
    ]
    f1s = [
        jax.lax.dot_general(
            a1, ht, (((0,), (0,)), ((), ())),
            preferred_element_type=jnp.float32)
        for ht in hts
    ]
    f2s = [
        jax.lax.dot_general(
            ht, a2, (((0,), (0,)), ((), ())),
            preferred_element_type=jnp.float32)
        for ht in hts
    ]
    atts = []
    for b in range(BB):
        et = f2s[b] + f1s[b]
        et = jnp.maximum(et, jnp.float32(ALPHA) * et)
        att = jnp.where(maskt, et, jnp.float32(-1e12))
        att = att - jnp.max(att, axis=1, keepdims=True)
        att = jnp.exp(att)
        atts.append(att / jnp.sum(att, axis=1, keepdims=True))
    for b in range(BB):
        hp = jax.lax.dot_general(
            hts[b], atts[b], (((1,), (0,)), ((), ())),
            preferred_element_type=jnp.float32)
        out_ref[b] = jnp.where(hp > 0, hp, jnp.exp(hp) - 1.0)


@jax.jit
def kernel(x, W, a, emb):
    grid = (B // BB,)
    xt = jnp.transpose(x, (0, 2, 1))
    return pl.pallas_call(
        _gat_kernel,
        grid=grid,
        in_specs=[
            pl.BlockSpec((BB, N, IN_FEAT), lambda b: (b, 0, 0)),
            pl.BlockSpec((IN_FEAT, OUT_FEAT), lambda b: (0, 0)),
            pl.BlockSpec((2 * OUT_FEAT, 1), lambda b: (0, 0)),
            pl.BlockSpec((N, EMBED_DIM), lambda b: (0, 0)),
        ],
        out_specs=pl.BlockSpec((BB, OUT_FEAT, N), lambda b: (b, 0, 0)),
        out_shape=jax.ShapeDtypeStruct((B, OUT_FEAT, N), jnp.float32),
        scratch_shapes=[pltpu.VMEM((N, N), jnp.float32)],
    )(xt, W, a, emb)


# speedup vs baseline: 1.2333x; 1.0472x over previous
import jax
import jax.numpy as jnp
from jax.experimental import pallas as pl
from jax.experimental.pallas import tpu as pltpu

B = 128
IN_FEAT = 256
OUT_FEAT = 128
N = 38
EMBED_DIM = 128
ALPHA = 0.2
TOP_K = 10

BB = 32
NP = 2 * N


def _gat_kernel(xp_ref, w_ref, a_ref, emb_ref, out_ref, maskp_ref):
    @pl.when(pl.program_id(0) == 0)
    def _():
        emb = emb_ref[...]
        gram = jax.lax.dot_general(
            emb, emb, (((1,), (1,)), ((), ())),
            preferred_element_type=jnp.float32)
        nrm = jnp.sqrt(jnp.sum(emb * emb, axis=1, keepdims=True))
        adj = gram / (nrm * nrm.T)
        a1_ = adj[:, None, :]
        a2_ = adj[None, :, :]
        mdx = jax.lax.broadcasted_iota(jnp.int32, (N, N, N), 0)
        kdx = jax.lax.broadcasted_iota(jnp.int32, (N, N, N), 1)
        gt = (a1_ > a2_) | ((a1_ == a2_) & (mdx < kdx))
        rank = jnp.sum(gt.astype(jnp.float32), axis=0)
        sel = (rank == jnp.float32(TOP_K - 2)).astype(jnp.float32)
        thresh_t = jnp.sum(adj * sel, axis=0, keepdims=True)
        maskt = ((adj > thresh_t) | (adj == jnp.float32(1.0))
                 ).astype(jnp.float32)
        tiled = jnp.tile(maskt, (2, 2))
        jp = jax.lax.broadcasted_iota(jnp.int32, (NP, NP), 0)
        ip = jax.lax.broadcasted_iota(jnp.int32, (NP, NP), 1)
        same = (jp < N) == (ip < N)
        maskp_ref[...] = jnp.where(same, tiled, jnp.float32(0.0))

    w = w_ref[...]
    a = a_ref[...]
    a1 = a[:OUT_FEAT, :]
    a2 = a[OUT_FEAT:, :]
    maskp = maskp_ref[...] > jnp.float32(0.5)

    NPAIR = BB // 2
    hts = [
        jax.lax.dot_general(
            w, jnp.transpose(xp_ref[p]), (((0,), (0,)), ((), ())),
            preferred_element_type=jnp.float32)
        for p in range(NPAIR)
    ]
    f1s = [
        jax.lax.dot_general(
            a1, ht, (((0,), (0,)), ((), ())),
            preferred_element_type=jnp.float32)
        for ht in hts
    ]
    f2s = [
        jax.lax.dot_general(
            ht, a2, (((0,), (0,)), ((), ())),
            preferred_element_type=jnp.float32)
        for ht in hts
    ]
    atts = []
    for p in range(NPAIR):
        et = f2s[p] + f1s[p]
        et = jnp.maximum(et, jnp.float32(ALPHA) * et)
        att = jnp.where(maskp, et, jnp.float32(-1e12))
        att = att - jnp.max(att, axis=1, keepdims=True)
        att = jnp.exp(att)
        atts.append(att / jnp.sum(att, axis=1, keepdims=True))
    for p in range(NPAIR):
        hp = jax.lax.dot_general(
            hts[p], atts[p], (((1,), (0,)), ((), ())),
            preferred_element_type=jnp.float32)
        o = jnp.where(hp > 0, hp, jnp.exp(hp) - 1.0)
        out_ref[2 * p] = o[:, :N]
        out_ref[2 * p + 1] = o[:, N:]


@jax.jit
def kernel(x, W, a, emb):
    xp = jnp.transpose(x, (0, 2, 1)).reshape(B // 2, NP, IN_FEAT)
    grid = (B // BB,)
    return pl.pallas_call(
        _gat_kernel,
        grid=grid,
        in_specs=[
            pl.BlockSpec((BB // 2, NP, IN_FEAT), lambda b: (b, 0, 0)),
            pl.BlockSpec((IN_FEAT, OUT_FEAT), lambda b: (0, 0)),
            pl.BlockSpec((2 * OUT_FEAT, 1), lambda b: (0, 0)),
            pl.BlockSpec((N, EMBED_DIM), lambda b: (0, 0)),
        ],
        out_specs=pl.BlockSpec((BB, OUT_FEAT, N), lambda b: (b, 0, 0)),
        out_shape=jax.ShapeDtypeStruct((B, OUT_FEAT, N), jnp.float32),
        scratch_shapes=[pltpu.VMEM((NP, NP), jnp.float32)],
    )(xp, W, a, emb)
